# baseline (speedup 1.0000x reference)
import functools

import jax
import jax.numpy as jnp
from jax import lax
from jax.experimental import pallas as pl
from jax.experimental.pallas import tpu as pltpu
from jax.experimental.pallas import tpu_sc as plsc

D = 32
NW = 32
L = 16
G_NBUF = 4
SK = 129


@functools.lru_cache(maxsize=None)
def _build_gather(batch: int, hist: int):
    nbt = batch // 128
    n_units = hist * nbt
    units_w = n_units // NW
    rounds = units_w // G_NBUF
    mesh = plsc.VectorSubcoreMesh(core_axis_name="c", subcore_axis_name="s")

    @functools.partial(
        pl.kernel,
        mesh=mesh,
        out_type=jax.ShapeDtypeStruct((hist, D // 8, nbt, 8, 128),
                                      jnp.float32),
        scratch_types=[
            pltpu.VMEM((G_NBUF, 128), jnp.int32),
            pltpu.VMEM((G_NBUF, 128, D), jnp.float32),
            pltpu.VMEM((G_NBUF, D // 8, 8, SK), jnp.float32),
            pltpu.SemaphoreType.DMA,
            pltpu.SemaphoreType.DMA,
            pltpu.SemaphoreType.DMA,
            pltpu.SemaphoreType.DMA,
            pltpu.SemaphoreType.DMA,
            pltpu.SemaphoreType.DMA,
            pltpu.SemaphoreType.DMA,
            pltpu.SemaphoreType.DMA,
            pltpu.SemaphoreType.DMA,
            pltpu.SemaphoreType.DMA,
            pltpu.SemaphoreType.DMA,
            pltpu.SemaphoreType.DMA,
        ],
        compiler_params=pltpu.CompilerParams(use_tc_tiling_on_sc=False,
                                             needs_layout_passes=False),
    )
    def gather_kernel(idxT_hbm, table_hbm, out_hbm, idx_v, rows_v, out_v,
                      *sems):
        gs = sems[:G_NBUF]
        ws = sems[G_NBUF:2 * G_NBUF]
        is_ = sems[2 * G_NBUF:3 * G_NBUF]
        w = lax.axis_index("s") * 2 + lax.axis_index("c")
        u0 = w * units_w
        lane = lax.iota(jnp.int32, L)
        dtv0 = lane // 8
        dtv1 = dtv0 + 2
        div = lane % 8

        def wb_copies(b, h, bt):
            return [
                pltpu.make_async_copy(
                    out_v.at[b, dt, :, pl.ds(0, 128)],
                    out_hbm.at[h, dt, bt], ws[b])
                for dt in range(D // 8)
            ]

        def round_(r, carry):
            us = [u0 + r * G_NBUF + b for b in range(G_NBUF)]
            hs = [u // nbt for u in us]
            bts = [u % nbt for u in us]
            for b in range(G_NBUF):
                pltpu.make_async_copy(
                    idxT_hbm.at[hs[b], pl.ds(bts[b] * 128, 128)],
                    idx_v.at[b], is_[b]).start()
            for b in range(G_NBUF):
                pltpu.make_async_copy(
                    idxT_hbm.at[hs[b], pl.ds(bts[b] * 128, 128)],
                    idx_v.at[b], is_[b]).wait()
                pltpu.async_copy(table_hbm.at[idx_v.at[b]],
                                 rows_v.at[b], gs[b])
            for b in range(G_NBUF):
                pltpu.make_async_copy(table_hbm.at[idx_v.at[b]],
                                      rows_v.at[b], gs[b]).wait()

                def jgroup(jg, carry2, b=b):
                    for jj in range(32):
                        j = jg * 32 + jj
                        jv = j + lane * 0
                        v0 = rows_v[b, j, pl.ds(0, L)]
                        v1 = rows_v[b, j, pl.ds(L, L)]
                        plsc.store_scatter(out_v.at[b], [dtv0, div, jv], v0)
                        plsc.store_scatter(out_v.at[b], [dtv1, div, jv], v1)
                    return carry2

                lax.fori_loop(0, 4, jgroup, 0)
            plsc.subcore_barrier()
            for b in range(G_NBUF):
                for cp in wb_copies(b, hs[b], bts[b]):
                    cp.start()
            for b in range(G_NBUF):
                for cp in wb_copies(b, hs[b], bts[b]):
                    cp.wait()
            return carry

        lax.fori_loop(0, rounds, round_, 0)

    return gather_kernel


def kernel(inputs, table):
    batch, hist = inputs.shape
    raw = _build_gather(batch, hist)(inputs.T, table)
    return raw.transpose(2, 4, 0, 1, 3).reshape(batch, hist, D)


# speedup vs baseline: 1.1477x; 1.1477x over previous
import functools

import jax
import jax.numpy as jnp
from jax import lax
from jax.experimental import pallas as pl
from jax.experimental.pallas import tpu as pltpu
from jax.experimental.pallas import tpu_sc as plsc

D = 32
NW = 32
L = 16
G_NBUF = 5
SK = 129


@functools.lru_cache(maxsize=None)
def _build_gather(batch: int, hist: int):
    nbt = batch // 128
    n_units = hist * nbt
    units_w = n_units // NW
    rounds = units_w // G_NBUF
    mesh = plsc.VectorSubcoreMesh(core_axis_name="c", subcore_axis_name="s")

    @functools.partial(
        pl.kernel,
        mesh=mesh,
        out_type=jax.ShapeDtypeStruct((hist, D // 8, nbt, 8, 128),
                                      jnp.float32),
        scratch_types=[
            pltpu.VMEM((G_NBUF, 128), jnp.int32),
            pltpu.VMEM((G_NBUF, 128, D), jnp.float32),
            pltpu.VMEM((G_NBUF, D // 8, 8, SK), jnp.float32),
            pltpu.SemaphoreType.DMA,
            pltpu.SemaphoreType.DMA,
            pltpu.SemaphoreType.DMA,
            pltpu.SemaphoreType.DMA,
            pltpu.SemaphoreType.DMA,
            pltpu.SemaphoreType.DMA,
            pltpu.SemaphoreType.DMA,
            pltpu.SemaphoreType.DMA,
            pltpu.SemaphoreType.DMA,
            pltpu.SemaphoreType.DMA,
            pltpu.SemaphoreType.DMA,
            pltpu.SemaphoreType.DMA,
            pltpu.SemaphoreType.DMA,
            pltpu.SemaphoreType.DMA,
            pltpu.SemaphoreType.DMA,
        ],
        compiler_params=pltpu.CompilerParams(use_tc_tiling_on_sc=False,
                                             needs_layout_passes=False),
    )
    def gather_kernel(idxT_hbm, table_hbm, out_hbm, idx_v, rows_v, out_v,
                      *sems):
        gs = sems[:G_NBUF]
        ws = sems[G_NBUF:2 * G_NBUF]
        is_ = sems[2 * G_NBUF:3 * G_NBUF]
        w = lax.axis_index("s") * 2 + lax.axis_index("c")
        u0 = w * units_w
        lane = lax.iota(jnp.int32, L)
        dtv0 = lane // 8
        dtv1 = dtv0 + 2
        div = lane % 8

        def wb_copies(b, h, bt):
            return [
                pltpu.make_async_copy(
                    out_v.at[b, dt, :, pl.ds(0, 128)],
                    out_hbm.at[h, dt, bt], ws[b])
                for dt in range(D // 8)
            ]

        def round_(r, carry):
            us = [u0 + r * G_NBUF + b for b in range(G_NBUF)]
            hs = [u // nbt for u in us]
            bts = [u % nbt for u in us]
            for b in range(G_NBUF):
                pltpu.make_async_copy(
                    idxT_hbm.at[hs[b], pl.ds(bts[b] * 128, 128)],
                    idx_v.at[b], is_[b]).start()
            for b in range(G_NBUF):
                pltpu.make_async_copy(
                    idxT_hbm.at[hs[b], pl.ds(bts[b] * 128, 128)],
                    idx_v.at[b], is_[b]).wait()
                pltpu.async_copy(table_hbm.at[idx_v.at[b]],
                                 rows_v.at[b], gs[b])
            for b in range(G_NBUF):
                pltpu.make_async_copy(table_hbm.at[idx_v.at[b]],
                                      rows_v.at[b], gs[b]).wait()

                def jgroup(jg, carry2, b=b):
                    for jj in range(8):
                        j = jg * 8 + jj
                        jv = j + lane * 0
                        v0 = rows_v[b, j, pl.ds(0, L)]
                        v1 = rows_v[b, j, pl.ds(L, L)]
                        plsc.store_scatter(out_v.at[b], [dtv0, div, jv], v0)
                        plsc.store_scatter(out_v.at[b], [dtv1, div, jv], v1)
                    return carry2

                lax.fori_loop(0, 16, jgroup, 0)
            plsc.subcore_barrier()
            for b in range(G_NBUF):
                for cp in wb_copies(b, hs[b], bts[b]):
                    cp.start()
            for b in range(G_NBUF):
                for cp in wb_copies(b, hs[b], bts[b]):
                    cp.wait()
            return carry

        lax.fori_loop(0, rounds, round_, 0)

    return gather_kernel


def kernel(inputs, table):
    batch, hist = inputs.shape
    raw = _build_gather(batch, hist)(inputs.T, table)
    return raw.transpose(2, 4, 0, 1, 3).reshape(batch, hist, D)


# speedup vs baseline: 1.1787x; 1.0271x over previous
import functools

import jax
import jax.numpy as jnp
from jax import lax
from jax.experimental import pallas as pl
from jax.experimental.pallas import tpu as pltpu
from jax.experimental.pallas import tpu_sc as plsc

D = 32
NW = 32
L = 16
G_NBUF = 8
SK = 129


@functools.lru_cache(maxsize=None)
def _build_gather(batch: int, hist: int):
    nbt = batch // 128
    n_units = hist * nbt
    units_w = n_units // NW
    rounds = units_w // G_NBUF
    mesh = plsc.VectorSubcoreMesh(core_axis_name="c", subcore_axis_name="s")

    @functools.partial(
        pl.kernel,
        mesh=mesh,
        out_type=jax.ShapeDtypeStruct((hist, D // 8, nbt, 8, 128),
                                      jnp.float32),
        scratch_types=[
            pltpu.VMEM((G_NBUF, 128), jnp.int32),
            pltpu.VMEM((G_NBUF, 128, D), jnp.float32),
            pltpu.VMEM((G_NBUF, D // 8, 8, SK), jnp.float32),
            pltpu.SemaphoreType.DMA,
            pltpu.SemaphoreType.DMA,
            pltpu.SemaphoreType.DMA,
            pltpu.SemaphoreType.DMA,
            pltpu.SemaphoreType.DMA,
            pltpu.SemaphoreType.DMA,
            pltpu.SemaphoreType.DMA,
            pltpu.SemaphoreType.DMA,
            pltpu.SemaphoreType.DMA,
            pltpu.SemaphoreType.DMA,
            pltpu.SemaphoreType.DMA,
            pltpu.SemaphoreType.DMA,
            pltpu.SemaphoreType.DMA,
            pltpu.SemaphoreType.DMA,
            pltpu.SemaphoreType.DMA,
            pltpu.SemaphoreType.DMA,
            pltpu.SemaphoreType.DMA,
            pltpu.SemaphoreType.DMA,
            pltpu.SemaphoreType.DMA,
            pltpu.SemaphoreType.DMA,
            pltpu.SemaphoreType.DMA,
            pltpu.SemaphoreType.DMA,
            pltpu.SemaphoreType.DMA,
            pltpu.SemaphoreType.DMA,
        ],
        compiler_params=pltpu.CompilerParams(use_tc_tiling_on_sc=False,
                                             needs_layout_passes=False),
    )
    def gather_kernel(idxT_hbm, table_hbm, out_hbm, idx_v, rows_v, out_v,
                      *sems):
        gs = sems[:G_NBUF]
        ws = sems[G_NBUF:2 * G_NBUF]
        is_ = sems[2 * G_NBUF:3 * G_NBUF]
        w = lax.axis_index("s") * 2 + lax.axis_index("c")
        u0 = w * units_w
        lane = lax.iota(jnp.int32, L)
        dtv0 = lane // 8
        dtv1 = dtv0 + 2
        div = lane % 8

        def wb_copies(b, h, bt):
            return [
                pltpu.make_async_copy(
                    out_v.at[b, dt, :, pl.ds(0, 128)],
                    out_hbm.at[h, dt, bt], ws[b])
                for dt in range(D // 8)
            ]

        def round_(r, carry):
            us = [u0 + r * G_NBUF + b for b in range(G_NBUF)]
            hs = [u // nbt for u in us]
            bts = [u % nbt for u in us]
            for b in range(G_NBUF):
                pltpu.make_async_copy(
                    idxT_hbm.at[hs[b], pl.ds(bts[b] * 128, 128)],
                    idx_v.at[b], is_[b]).start()
            for b in range(G_NBUF):
                pltpu.make_async_copy(
                    idxT_hbm.at[hs[b], pl.ds(bts[b] * 128, 128)],
                    idx_v.at[b], is_[b]).wait()
                pltpu.async_copy(table_hbm.at[idx_v.at[b]],
                                 rows_v.at[b], gs[b])
            for b in range(G_NBUF):
                pltpu.make_async_copy(table_hbm.at[idx_v.at[b]],
                                      rows_v.at[b], gs[b]).wait()

                def jgroup(jg, carry2, b=b):
                    for jj in range(8):
                        j = jg * 8 + jj
                        jv = j + lane * 0
                        v0 = rows_v[b, j, pl.ds(0, L)]
                        v1 = rows_v[b, j, pl.ds(L, L)]
                        plsc.store_scatter(out_v.at[b], [dtv0, div, jv], v0)
                        plsc.store_scatter(out_v.at[b], [dtv1, div, jv], v1)
                    return carry2

                lax.fori_loop(0, 16, jgroup, 0)
            plsc.subcore_barrier()
            for b in range(G_NBUF):
                for cp in wb_copies(b, hs[b], bts[b]):
                    cp.start()
            for b in range(G_NBUF):
                for cp in wb_copies(b, hs[b], bts[b]):
                    cp.wait()
            return carry

        lax.fori_loop(0, rounds, round_, 0)

    return gather_kernel


def kernel(inputs, table):
    batch, hist = inputs.shape
    raw = _build_gather(batch, hist)(inputs.T, table)
    return raw.transpose(2, 4, 0, 1, 3).reshape(batch, hist, D)


# speedup vs baseline: 1.1998x; 1.0178x over previous
import functools

import jax
import jax.numpy as jnp
from jax import lax
from jax.experimental import pallas as pl
from jax.experimental.pallas import tpu as pltpu
from jax.experimental.pallas import tpu_sc as plsc

D = 32
NW = 32
L = 16
G_NBUF = 10
SK = 129


@functools.lru_cache(maxsize=None)
def _build_gather(batch: int, hist: int):
    nbt = batch // 128
    n_units = hist * nbt
    units_w = n_units // NW
    rounds = units_w // G_NBUF
    mesh = plsc.VectorSubcoreMesh(core_axis_name="c", subcore_axis_name="s")

    @functools.partial(
        pl.kernel,
        mesh=mesh,
        out_type=jax.ShapeDtypeStruct((hist, D // 8, nbt, 8, 128),
                                      jnp.float32),
        scratch_types=[
            pltpu.VMEM((G_NBUF, 128), jnp.int32),
            pltpu.VMEM((G_NBUF, 128, D), jnp.float32),
            pltpu.VMEM((G_NBUF, D // 8, 8, SK), jnp.float32),
            pltpu.SemaphoreType.DMA,
            pltpu.SemaphoreType.DMA,
            pltpu.SemaphoreType.DMA,
            pltpu.SemaphoreType.DMA,
            pltpu.SemaphoreType.DMA,
            pltpu.SemaphoreType.DMA,
            pltpu.SemaphoreType.DMA,
            pltpu.SemaphoreType.DMA,
            pltpu.SemaphoreType.DMA,
            pltpu.SemaphoreType.DMA,
            pltpu.SemaphoreType.DMA,
            pltpu.SemaphoreType.DMA,
            pltpu.SemaphoreType.DMA,
            pltpu.SemaphoreType.DMA,
            pltpu.SemaphoreType.DMA,
            pltpu.SemaphoreType.DMA,
            pltpu.SemaphoreType.DMA,
            pltpu.SemaphoreType.DMA,
            pltpu.SemaphoreType.DMA,
            pltpu.SemaphoreType.DMA,
            pltpu.SemaphoreType.DMA,
            pltpu.SemaphoreType.DMA,
            pltpu.SemaphoreType.DMA,
            pltpu.SemaphoreType.DMA,
            pltpu.SemaphoreType.DMA,
            pltpu.SemaphoreType.DMA,
            pltpu.SemaphoreType.DMA,
            pltpu.SemaphoreType.DMA,
            pltpu.SemaphoreType.DMA,
            pltpu.SemaphoreType.DMA,
        ],
        compiler_params=pltpu.CompilerParams(use_tc_tiling_on_sc=False,
                                             needs_layout_passes=False),
    )
    def gather_kernel(idxT_hbm, table_hbm, out_hbm, idx_v, rows_v, out_v,
                      *sems):
        gs = sems[:G_NBUF]
        ws = sems[G_NBUF:2 * G_NBUF]
        is_ = sems[2 * G_NBUF:3 * G_NBUF]
        w = lax.axis_index("s") * 2 + lax.axis_index("c")
        u0 = w * units_w
        lane = lax.iota(jnp.int32, L)
        dtv0 = lane // 8
        dtv1 = dtv0 + 2
        div = lane % 8

        def wb_copies(b, h, bt):
            return [
                pltpu.make_async_copy(
                    out_v.at[b, dt, :, pl.ds(0, 128)],
                    out_hbm.at[h, dt, bt], ws[b])
                for dt in range(D // 8)
            ]

        def round_(r, carry):
            us = [u0 + r * G_NBUF + b for b in range(G_NBUF)]
            hs = [u // nbt for u in us]
            bts = [u % nbt for u in us]
            for b in range(G_NBUF):
                pltpu.make_async_copy(
                    idxT_hbm.at[hs[b], pl.ds(bts[b] * 128, 128)],
                    idx_v.at[b], is_[b]).start()
            for b in range(G_NBUF):
                pltpu.make_async_copy(
                    idxT_hbm.at[hs[b], pl.ds(bts[b] * 128, 128)],
                    idx_v.at[b], is_[b]).wait()
                pltpu.async_copy(table_hbm.at[idx_v.at[b]],
                                 rows_v.at[b], gs[b])
            for b in range(G_NBUF):
                pltpu.make_async_copy(table_hbm.at[idx_v.at[b]],
                                      rows_v.at[b], gs[b]).wait()

                def jgroup(jg, carry2, b=b):
                    for jj in range(8):
                        j = jg * 8 + jj
                        jv = j + lane * 0
                        v0 = rows_v[b, j, pl.ds(0, L)]
                        v1 = rows_v[b, j, pl.ds(L, L)]
                        plsc.store_scatter(out_v.at[b], [dtv0, div, jv], v0)
                        plsc.store_scatter(out_v.at[b], [dtv1, div, jv], v1)
                    return carry2

                lax.fori_loop(0, 16, jgroup, 0)
            plsc.subcore_barrier()
            for b in range(G_NBUF):
                for cp in wb_copies(b, hs[b], bts[b]):
                    cp.start()
            for b in range(G_NBUF):
                for cp in wb_copies(b, hs[b], bts[b]):
                    cp.wait()
            return carry

        lax.fori_loop(0, rounds, round_, 0)

    return gather_kernel


def kernel(inputs, table):
    batch, hist = inputs.shape
    raw = _build_gather(batch, hist)(inputs.T, table)
    return raw.transpose(2, 4, 0, 1, 3).reshape(batch, hist, D)
